# baseline (speedup 1.0000x reference)
import functools

import jax
import jax.numpy as jnp
from jax import lax
from jax.experimental import pallas as pl
from jax.experimental.pallas import tpu as pltpu
from jax.experimental.pallas import tpu_sc as plsc

N = 10000
E = 160000
H = 32
HE = 64
DEMB = 256
NG = 64
EPS = 1e-5

NC = 2
NS = 16
NW = NC * NS
EPW = E // NW
CH = 1000
NCH = EPW // CH
RPT = N // NS

EB = 640
_SC_MESH = plsc.VectorSubcoreMesh(core_axis_name="c", subcore_axis_name="s")
_SC_PARAMS = pltpu.CompilerParams(use_tc_tiling_on_sc=False)


@functools.partial(
    pl.kernel,
    out_type=jax.ShapeDtypeStruct((E, H), jnp.float32),
    mesh=_SC_MESH,
    scratch_types=[
        pltpu.VMEM((CH,), jnp.int32),
        pltpu.VMEM((CH, H), jnp.float32),
        pltpu.SemaphoreType.DMA,
    ],
    compiler_params=_SC_PARAMS,
)
def _sc_gather(table_hbm, idx_hbm, out_hbm, idx_v, rows_v, sem):
    wid = lax.axis_index("c") * NS + lax.axis_index("s")
    for j in range(NCH):
        base = pl.multiple_of(wid * EPW + j * CH, 8)
        pltpu.sync_copy(idx_hbm.at[pl.ds(base, CH)], idx_v)
        pltpu.async_copy(table_hbm.at[idx_v], rows_v, sem).wait()
        pltpu.sync_copy(rows_v, out_hbm.at[pl.ds(base, CH)])


@functools.partial(
    pl.kernel,
    out_type=jax.ShapeDtypeStruct((NC, N, H), jnp.float32),
    mesh=_SC_MESH,
    scratch_types=[
        pltpu.VMEM((CH,), jnp.int32),
        pltpu.VMEM((CH, H), jnp.float32),
        pltpu.VMEM_SHARED((N, H), jnp.float32),
    ],
    compiler_params=_SC_PARAMS,
)
def _sc_scatter(msg_hbm, dst_hbm, zeros_hbm, out_hbm, idx_v, rows_v, agg_sh):
    c = lax.axis_index("c")
    s = lax.axis_index("s")
    wid = c * NS + s
    row0 = s * RPT
    pltpu.sync_copy(zeros_hbm.at[pl.ds(row0, RPT)], agg_sh.at[pl.ds(row0, RPT)])
    plsc.subcore_barrier()
    for j in range(NCH):
        base = pl.multiple_of(wid * EPW + j * CH, 8)
        pltpu.sync_copy(dst_hbm.at[pl.ds(base, CH)], idx_v)
        pltpu.sync_copy(msg_hbm.at[pl.ds(base, CH)], rows_v)
        pltpu.sync_copy(rows_v, agg_sh.at[idx_v], add=True)
    plsc.subcore_barrier()
    pltpu.sync_copy(agg_sh.at[pl.ds(row0, RPT)], out_hbm.at[c, pl.ds(row0, RPT)])


E4 = E // 4
EB4 = 1000


def _edge_body(ea, xj, w1b, b1b, w2b, b2bm, repb, foldb, out):
    h1 = jnp.maximum(
        jnp.dot(ea[...], w1b[...], preferred_element_type=jnp.float32) + b1b[...],
        0.0,
    )
    w = jnp.dot(h1.astype(jnp.bfloat16), w2b[...],
                preferred_element_type=jnp.float32)
    xjh = xj[...].astype(jnp.bfloat16)
    xr = jnp.dot(xjh, repb[...], preferred_element_type=jnp.float32)
    p = w * xr
    accs = []
    for u in range(4):
        acc = p[:, 1024 * u:1024 * u + 128]
        for j in range(1, 8):
            lo = 1024 * u + 128 * j
            acc = acc + p[:, lo:lo + 128]
        accs.append(acc)
    acc_cat = jnp.concatenate(accs, axis=1)
    out[...] = (jnp.dot(acc_cat, foldb[...], preferred_element_type=jnp.float32)
                + jnp.dot(xjh, b2bm[...], preferred_element_type=jnp.float32))


def _edge_messages(ea, xj, w1b, b1b, w2b, b2bm, repb, foldb):
    grid = (E4 // EB4,)
    return pl.pallas_call(
        _edge_body,
        grid=grid,
        in_specs=[
            pl.BlockSpec((EB4, 128), lambda i: (i, 0)),
            pl.BlockSpec((EB4, 128), lambda i: (i, 0)),
            pl.BlockSpec((128, 4 * HE), lambda i: (0, 0)),
            pl.BlockSpec((1, 4 * HE), lambda i: (0, 0)),
            pl.BlockSpec((4 * HE, 4 * H * H), lambda i: (0, 0)),
            pl.BlockSpec((128, 128), lambda i: (0, 0)),
            pl.BlockSpec((128, 4 * H * H), lambda i: (0, 0)),
            pl.BlockSpec((512, 128), lambda i: (0, 0)),
        ],
        out_specs=pl.BlockSpec((EB4, 128), lambda i: (i, 0)),
        out_shape=jax.ShapeDtypeStruct((E4, 128), jnp.float32),
    )(ea, xj, w1b, b1b, w2b, b2bm, repb, foldb)


def _node_body(aggp, h_in, root, bias, gamma, beta, out):
    agg = aggp[0] + aggp[1]
    t = agg + jnp.dot(h_in[...], root[...], preferred_element_type=jnp.float32)
    t = t + bias[...]
    mu = jnp.mean(t, axis=0, keepdims=True)
    var = jnp.mean((t - mu) ** 2, axis=0, keepdims=True)
    bn = gamma[...] * (t - mu) * lax.rsqrt(var + EPS) + beta[...]
    out[...] = jnp.maximum(bn, 0.0)


def _node_update(aggp, h_in, root, bias, gamma, beta):
    return pl.pallas_call(
        _node_body,
        out_shape=jax.ShapeDtypeStruct((N, H), jnp.float32),
    )(aggp, h_in, root, bias, gamma, beta)


def _final_body(aggp, h_in, root, bias, gamma, beta, batch2d, wro, bro, out):
    agg = aggp[0] + aggp[1]
    t = agg + jnp.dot(h_in[...], root[...], preferred_element_type=jnp.float32)
    t = t + bias[...]
    mu = jnp.mean(t, axis=0, keepdims=True)
    var = jnp.mean((t - mu) ** 2, axis=0, keepdims=True)
    bn = gamma[...] * (t - mu) * lax.rsqrt(var + EPS) + beta[...]
    h = jnp.maximum(bn, 0.0)
    gids = lax.broadcasted_iota(jnp.int32, (NG, N), 0)
    onehot = (gids == batch2d[...]).astype(jnp.float32)
    pooled = jnp.dot(onehot, h, preferred_element_type=jnp.float32)
    out[...] = jnp.dot(pooled, wro[...], preferred_element_type=jnp.float32) + bro[...]


def _final_update(aggp, h_in, root, bias, gamma, beta, batch2d, wro, bro):
    return pl.pallas_call(
        _final_body,
        out_shape=jax.ShapeDtypeStruct((NG, DEMB), jnp.float32),
    )(aggp, h_in, root, bias, gamma, beta, batch2d, wro, bro)


def kernel(x, edge_index, edge_attr, batch, ew1, eb1, ew2, eb2,
           root0, bias0, root1, bias1, gamma0, beta0, gamma1, beta1, Wro, bro):
    src = edge_index[0]
    dst = edge_index[1]
    eb1r = eb1.reshape(1, HE)
    eb2r = eb2.reshape(1, H * H)
    bias0r = bias0.reshape(1, H)
    bias1r = bias1.reshape(1, H)
    gamma0r = gamma0.reshape(1, H)
    gamma1r = gamma1.reshape(1, H)
    beta0r = beta0.reshape(1, H)
    beta1r = beta1.reshape(1, H)
    bror = bro.reshape(1, DEMB)
    batch2d = batch.reshape(1, N)
    zeros_nh = jnp.zeros((N, H), jnp.float32)
    rep = jnp.repeat(jnp.eye(H, dtype=jnp.bfloat16), H, axis=1)
    fold = jnp.tile(jnp.eye(H, dtype=jnp.float32), (4, 1))
    bd = jax.scipy.linalg.block_diag
    w1b = bd(ew1, ew1, ew1, ew1)
    b1b = jnp.tile(eb1r, (1, 4))
    ew2h = ew2.astype(jnp.bfloat16)
    w2b = bd(ew2h, ew2h, ew2h, ew2h)
    b2m = eb2.reshape(H, H).astype(jnp.bfloat16)
    b2bm = bd(b2m, b2m, b2m, b2m)
    repb = bd(rep, rep, rep, rep)
    foldb = bd(fold, fold, fold, fold)
    eaP = edge_attr.reshape(E4, 128)

    xj0 = _sc_gather(x, src)
    msg0 = _edge_messages(eaP, xj0.reshape(E4, 128),
                          w1b, b1b, w2b, b2bm, repb, foldb).reshape(E, H)
    agg0 = _sc_scatter(msg0, dst, zeros_nh)
    h1 = _node_update(agg0, x, root0, bias0r, gamma0r, beta0r)

    xj1 = _sc_gather(h1, src)
    msg1 = _edge_messages(eaP, xj1.reshape(E4, 128),
                          w1b, b1b, w2b, b2bm, repb, foldb).reshape(E, H)
    agg1 = _sc_scatter(msg1, dst, zeros_nh)
    return _final_update(agg1, h1, root1, bias1r, gamma1r, beta1r,
                         batch2d, Wro, bror)


# speedup vs baseline: 1.0472x; 1.0472x over previous
import functools

import jax
import jax.numpy as jnp
from jax import lax
from jax.experimental import pallas as pl
from jax.experimental.pallas import tpu as pltpu
from jax.experimental.pallas import tpu_sc as plsc

N = 10000
E = 160000
H = 32
HE = 64
DEMB = 256
NG = 64
EPS = 1e-5

NC = 2
NS = 16
NW = NC * NS
EPW = E // NW
CH = 1000
NCH = EPW // CH
RPT = N // NS

EB = 640
_SC_MESH = plsc.VectorSubcoreMesh(core_axis_name="c", subcore_axis_name="s")
_SC_PARAMS = pltpu.CompilerParams(use_tc_tiling_on_sc=False)


@functools.partial(
    pl.kernel,
    out_type=jax.ShapeDtypeStruct((E, H), jnp.float32),
    mesh=_SC_MESH,
    scratch_types=[
        pltpu.VMEM((CH,), jnp.int32),
        pltpu.VMEM((CH,), jnp.int32),
        pltpu.VMEM((CH, H), jnp.float32),
        pltpu.VMEM((CH, H), jnp.float32),
        pltpu.SemaphoreType.DMA,
        pltpu.SemaphoreType.DMA,
    ],
    compiler_params=_SC_PARAMS,
)
def _sc_gather(table_hbm, ei_hbm, out_hbm, idx_a, idx_b, rows_a, rows_b,
               sem_a, sem_b):
    wid = lax.axis_index("c") * NS + lax.axis_index("s")
    bufs = [(idx_a, rows_a, sem_a), (idx_b, rows_b, sem_b)]
    pend = [None, None]
    pend_base = [None, None]
    for j in range(NCH):
        b = j % 2
        if pend[b] is not None:
            pend[b].wait()
            pltpu.sync_copy(bufs[b][1], out_hbm.at[pl.ds(pend_base[b], CH)])
        base = pl.multiple_of(wid * EPW + j * CH, 8)
        pltpu.sync_copy(ei_hbm.at[0, pl.ds(base, CH)], bufs[b][0])
        pend[b] = pltpu.async_copy(table_hbm.at[bufs[b][0]], bufs[b][1],
                                   bufs[b][2])
        pend_base[b] = base
    for j in (NCH - 1, NCH):
        b = j % 2
        if pend[b] is not None:
            pend[b].wait()
            pltpu.sync_copy(bufs[b][1], out_hbm.at[pl.ds(pend_base[b], CH)])
            pend[b] = None


@functools.partial(
    pl.kernel,
    out_type=jax.ShapeDtypeStruct((NC, N, H), jnp.float32),
    mesh=_SC_MESH,
    scratch_types=[
        pltpu.VMEM((CH,), jnp.int32),
        pltpu.VMEM((CH,), jnp.int32),
        pltpu.VMEM((CH, H), jnp.float32),
        pltpu.VMEM((CH, H), jnp.float32),
        pltpu.VMEM_SHARED((N, H), jnp.float32),
        pltpu.SemaphoreType.DMA,
        pltpu.SemaphoreType.DMA,
        pltpu.SemaphoreType.DMA,
        pltpu.SemaphoreType.DMA,
    ],
    compiler_params=_SC_PARAMS,
)
def _sc_scatter(msg_hbm, ei_hbm, zeros_hbm, out_hbm, idx_a, idx_b,
                rows_a, rows_b, agg_sh, si_a, si_b, sr_a, sr_b):
    c = lax.axis_index("c")
    s = lax.axis_index("s")
    wid = c * NS + s
    row0 = s * RPT
    idxs = [idx_a, idx_b]
    rows = [rows_a, rows_b]
    sems = [(si_a, sr_a), (si_b, sr_b)]

    def start_loads(j):
        b = j % 2
        base = pl.multiple_of(wid * EPW + j * CH, 8)
        di = pltpu.async_copy(ei_hbm.at[1, pl.ds(base, CH)], idxs[b],
                              sems[b][0])
        dr = pltpu.async_copy(msg_hbm.at[pl.ds(base, CH)], rows[b],
                              sems[b][1])
        return di, dr

    pend = start_loads(0)
    pltpu.sync_copy(zeros_hbm.at[pl.ds(row0, RPT)], agg_sh.at[pl.ds(row0, RPT)])
    plsc.subcore_barrier()
    for j in range(NCH):
        nxt = start_loads(j + 1) if j + 1 < NCH else None
        pend[0].wait()
        pend[1].wait()
        b = j % 2
        pltpu.sync_copy(rows[b], agg_sh.at[idxs[b]], add=True)
        pend = nxt
    plsc.subcore_barrier()
    pltpu.sync_copy(agg_sh.at[pl.ds(row0, RPT)], out_hbm.at[c, pl.ds(row0, RPT)])


E4 = E // 4
EB4 = 800


def _edge_body(ea, xj, w1b, b1b, w2b, b2bm, repb, foldb, out):
    h1 = jnp.maximum(
        jnp.dot(ea[...], w1b[...], preferred_element_type=jnp.float32) + b1b[...],
        0.0,
    )
    w = jnp.dot(h1.astype(jnp.bfloat16), w2b[...],
                preferred_element_type=jnp.float32)
    xjh = xj[...].astype(jnp.bfloat16)
    xr = jnp.dot(xjh, repb[...], preferred_element_type=jnp.float32)
    p = w * xr
    accs = []
    for u in range(4):
        acc = p[:, 1024 * u:1024 * u + 128]
        for j in range(1, 8):
            lo = 1024 * u + 128 * j
            acc = acc + p[:, lo:lo + 128]
        accs.append(acc)
    acc_cat = jnp.concatenate(accs, axis=1)
    out[...] = (jnp.dot(acc_cat, foldb[...], preferred_element_type=jnp.float32)
                + jnp.dot(xjh, b2bm[...], preferred_element_type=jnp.float32))


def _edge_messages(ea, xj, w1b, b1b, w2b, b2bm, repb, foldb):
    grid = (E4 // EB4,)
    return pl.pallas_call(
        _edge_body,
        grid=grid,
        in_specs=[
            pl.BlockSpec((EB4, 128), lambda i: (i, 0)),
            pl.BlockSpec((EB4, 128), lambda i: (i, 0)),
            pl.BlockSpec((128, 4 * HE), lambda i: (0, 0)),
            pl.BlockSpec((1, 4 * HE), lambda i: (0, 0)),
            pl.BlockSpec((4 * HE, 4 * H * H), lambda i: (0, 0)),
            pl.BlockSpec((128, 128), lambda i: (0, 0)),
            pl.BlockSpec((128, 4 * H * H), lambda i: (0, 0)),
            pl.BlockSpec((512, 128), lambda i: (0, 0)),
        ],
        out_specs=pl.BlockSpec((EB4, 128), lambda i: (i, 0)),
        out_shape=jax.ShapeDtypeStruct((E4, 128), jnp.float32),
    )(ea, xj, w1b, b1b, w2b, b2bm, repb, foldb)


def _node_body(aggp, h_in, root, bias, gamma, beta, out):
    agg = aggp[0] + aggp[1]
    t = agg + jnp.dot(h_in[...], root[...], preferred_element_type=jnp.float32)
    t = t + bias[...]
    mu = jnp.mean(t, axis=0, keepdims=True)
    var = jnp.mean((t - mu) ** 2, axis=0, keepdims=True)
    bn = gamma[...] * (t - mu) * lax.rsqrt(var + EPS) + beta[...]
    out[...] = jnp.maximum(bn, 0.0)


def _node_update(aggp, h_in, root, bias, gamma, beta):
    return pl.pallas_call(
        _node_body,
        out_shape=jax.ShapeDtypeStruct((N, H), jnp.float32),
    )(aggp, h_in, root, bias, gamma, beta)


def _final_body(aggp, h_in, root, bias, gamma, beta, batch2d, wro, bro, out):
    agg = aggp[0] + aggp[1]
    t = agg + jnp.dot(h_in[...], root[...], preferred_element_type=jnp.float32)
    t = t + bias[...]
    mu = jnp.mean(t, axis=0, keepdims=True)
    var = jnp.mean((t - mu) ** 2, axis=0, keepdims=True)
    bn = gamma[...] * (t - mu) * lax.rsqrt(var + EPS) + beta[...]
    h = jnp.maximum(bn, 0.0)
    gids = lax.broadcasted_iota(jnp.int32, (NG, N), 0)
    onehot = (gids == batch2d[...]).astype(jnp.float32)
    pooled = jnp.dot(onehot, h, preferred_element_type=jnp.float32)
    out[...] = jnp.dot(pooled, wro[...], preferred_element_type=jnp.float32) + bro[...]


def _final_update(aggp, h_in, root, bias, gamma, beta, batch2d, wro, bro):
    return pl.pallas_call(
        _final_body,
        out_shape=jax.ShapeDtypeStruct((NG, DEMB), jnp.float32),
    )(aggp, h_in, root, bias, gamma, beta, batch2d, wro, bro)


def kernel(x, edge_index, edge_attr, batch, ew1, eb1, ew2, eb2,
           root0, bias0, root1, bias1, gamma0, beta0, gamma1, beta1, Wro, bro):
    eb1r = eb1.reshape(1, HE)
    eb2r = eb2.reshape(1, H * H)
    bias0r = bias0.reshape(1, H)
    bias1r = bias1.reshape(1, H)
    gamma0r = gamma0.reshape(1, H)
    gamma1r = gamma1.reshape(1, H)
    beta0r = beta0.reshape(1, H)
    beta1r = beta1.reshape(1, H)
    bror = bro.reshape(1, DEMB)
    batch2d = batch.reshape(1, N)
    zeros_nh = jnp.zeros((N, H), jnp.float32)
    rep = jnp.repeat(jnp.eye(H, dtype=jnp.bfloat16), H, axis=1)
    fold = jnp.tile(jnp.eye(H, dtype=jnp.float32), (4, 1))
    bd = jax.scipy.linalg.block_diag
    w1b = bd(ew1, ew1, ew1, ew1)
    b1b = jnp.tile(eb1r, (1, 4))
    ew2h = ew2.astype(jnp.bfloat16)
    w2b = bd(ew2h, ew2h, ew2h, ew2h)
    b2m = eb2.reshape(H, H).astype(jnp.bfloat16)
    b2bm = bd(b2m, b2m, b2m, b2m)
    repb = bd(rep, rep, rep, rep)
    foldb = bd(fold, fold, fold, fold)
    eaP = edge_attr.reshape(E4, 128)

    xj0 = _sc_gather(x, edge_index)
    msg0 = _edge_messages(eaP, xj0.reshape(E4, 128),
                          w1b, b1b, w2b, b2bm, repb, foldb).reshape(E, H)
    agg0 = _sc_scatter(msg0, edge_index, zeros_nh)
    h1 = _node_update(agg0, x, root0, bias0r, gamma0r, beta0r)

    xj1 = _sc_gather(h1, edge_index)
    msg1 = _edge_messages(eaP, xj1.reshape(E4, 128),
                          w1b, b1b, w2b, b2bm, repb, foldb).reshape(E, H)
    agg1 = _sc_scatter(msg1, edge_index, zeros_nh)
    return _final_update(agg1, h1, root1, bias1r, gamma1r, beta1r,
                         batch2d, Wro, bror)
